# baseline (speedup 1.0000x reference)
import functools

import jax
import jax.numpy as jnp
from jax import lax
from jax.experimental import pallas as pl
from jax.experimental.pallas import tpu as pltpu
from jax.experimental.pallas import tpu_sc as plsc

_N = 10000
_D = 128
_B = 128
_K = 30
_NC = 2
_NS = 16
_NW = _NC * _NS
_CH = 128
_NP = 10240
_RPT = _NP // _NS
_MP = 10112
_ZROW = _MP - 1
_NEG = -3.0e38
_BIGI = 1 << 29
_PADG = 1 << 20
_IDB = 16
_ECH = 64
_EIDB = 32
_NBUF = 4
_AGG_FRAC0 = 100


def _mesh():
  return plsc.VectorSubcoreMesh(core_axis_name="c", subcore_axis_name="s",
                                num_cores=_NC, num_subcores=_NS)


@functools.cache
def _make_sc_agg(nb0: int, nb1: int):
  cpt0 = nb0 * _EIDB
  cpt1 = nb1 * _EIDB
  scratch = [
      pltpu.VMEM((_EIDB, _ECH), jnp.int32),
      pltpu.VMEM((_EIDB, _ECH), jnp.int32),
      pltpu.VMEM_SHARED((_NP, _D), jnp.float32),
  ] + [pltpu.VMEM((_ECH, _D), jnp.float32)] * _NBUF \
    + [pltpu.SemaphoreType.DMA] * (2 * _NBUF)

  def body(h_hbm, src_hbm, dst_hbm, zeros_hbm, out_acc,
           src_v, dst_v, acc_sh, *bufs):
    rows = bufs[:_NBUF]
    gsem = bufs[_NBUF:2 * _NBUF]
    ssem = bufs[2 * _NBUF:]
    c = lax.axis_index("c")
    s = lax.axis_index("s")
    pltpu.sync_copy(zeros_hbm.at[pl.ds(s * _RPT, _RPT)],
                    acc_sh.at[pl.ds(s * _RPT, _RPT)])
    nblk = jnp.where(c == 0, nb0, nb1)
    t0 = jnp.where(c == 0, s * cpt0, _NS * cpt0 + s * cpt1)
    plsc.subcore_barrier()

    def block(bi, carry):
      pltpu.sync_copy(src_hbm.at[pl.ds(t0 + bi * _EIDB, _EIDB)], src_v)
      pltpu.sync_copy(dst_hbm.at[pl.ds(t0 + bi * _EIDB, _EIDB)], dst_v)
      gd = [None] * _NBUF
      sd = [None] * _NBUF
      for p in range(_NBUF - 1):
        gd[p] = pltpu.async_copy(h_hbm.at[src_v.at[p]], rows[p], gsem[p])
      for j in range(_EIDB):
        b = j % _NBUF
        nxt = j + _NBUF - 1
        if nxt < _EIDB:
          nb = nxt % _NBUF
          if sd[nb] is not None:
            sd[nb].wait()
          gd[nb] = pltpu.async_copy(h_hbm.at[src_v.at[nxt]],
                                    rows[nb], gsem[nb])
        gd[b].wait()
        sd[b] = pltpu.async_copy(rows[b], acc_sh.at[dst_v.at[j]],
                                 ssem[b], add=True)
      for p in range(_NBUF):
        sd[(_EIDB - _NBUF + p) % _NBUF].wait()
      return carry

    lax.fori_loop(0, nblk, block, 0)
    plsc.subcore_barrier()
    off = c * _NP + s * _RPT
    pltpu.sync_copy(acc_sh.at[pl.ds(s * _RPT, _RPT)],
                    out_acc.at[pl.ds(off, _RPT)])

  return pl.kernel(
      body,
      out_type=[jax.ShapeDtypeStruct((_NC * _NP, _D), jnp.float32)],
      mesh=_mesh(), scratch_types=scratch)


@functools.cache
def _make_sc_deg(chunks_per_tile: int):
  scratch = [
      pltpu.VMEM((_IDB, _CH), jnp.int32),
      pltpu.VMEM((_CH, _D), jnp.float32),
      pltpu.VMEM_SHARED((_NP, _D), jnp.float32),
      pltpu.SemaphoreType.DMA,
  ]

  def body(dst_hbm, zeros_hbm, ones_hbm, out_acc, dst_v, ones_v, acc_sh, sem):
    c = lax.axis_index("c")
    s = lax.axis_index("s")
    pltpu.sync_copy(zeros_hbm.at[pl.ds(s * _RPT, _RPT)],
                    acc_sh.at[pl.ds(s * _RPT, _RPT)])
    pltpu.sync_copy(ones_hbm, ones_v)
    t0 = (c * _NS + s) * chunks_per_tile
    plsc.subcore_barrier()

    def block(bi, carry):
      pltpu.sync_copy(dst_hbm.at[pl.ds(t0 + bi * _IDB, _IDB)], dst_v)
      descs = [pltpu.async_copy(ones_v, acc_sh.at[dst_v.at[j]], sem, add=True)
               for j in range(_IDB)]
      for d in descs:
        d.wait()
      return carry

    lax.fori_loop(0, chunks_per_tile // _IDB, block, 0)
    plsc.subcore_barrier()
    off = c * _NP + s * _RPT
    pltpu.sync_copy(acc_sh.at[pl.ds(s * _RPT, _RPT)],
                    out_acc.at[pl.ds(off, _RPT)])

  return pl.kernel(
      body,
      out_type=[jax.ShapeDtypeStruct((_NC * _NP, _D), jnp.float32)],
      mesh=_mesh(), scratch_types=scratch)


def _sc_gather_body(h_hbm, sel_hbm, out_hbm, idx_v, rows_v, sem):
  w = lax.axis_index("c") * _NS + lax.axis_index("s")
  pltpu.sync_copy(sel_hbm.at[pl.ds(w * _CH, _CH)], idx_v)
  pltpu.async_copy(h_hbm.at[idx_v], rows_v, sem).wait()
  pltpu.sync_copy(rows_v, out_hbm.at[pl.ds(w * _CH, _CH)])


@functools.cache
def _make_sc_gather():
  return pl.kernel(
      _sc_gather_body,
      out_type=[jax.ShapeDtypeStruct((_NW * _CH, _D), jnp.float32)],
      mesh=_mesh(),
      scratch_types=[
          pltpu.VMEM((_CH,), jnp.int32),
          pltpu.VMEM((_CH, _D), jnp.float32),
          pltpu.SemaphoreType.DMA,
      ])


_R = 400
_GRID = _N // _R


def _dense1_body(parts_ref, degp_ref, h_ref, wlt_ref, b_ref, wrt_ref,
                 out_ref, dinv_ref):
  agg = parts_ref[0] + parts_ref[1]
  deg = degp_ref[0, :, 0:1] + degp_ref[1, :, 0:1]
  dinv = 1.0 / jnp.maximum(deg, 1.0)
  mean = agg * dinv
  o = (jnp.dot(mean, wlt_ref[...], preferred_element_type=jnp.float32)
       + b_ref[0:1, :]
       + jnp.dot(h_ref[...], wrt_ref[...], preferred_element_type=jnp.float32))
  out_ref[...] = jnp.maximum(o, 0.0)
  dinv_ref[...] = jnp.broadcast_to(dinv, (_R, _D))


def _dense23_body(parts_ref, dinv_ref, h_ref, wlt_ref, b_ref, wrt_ref,
                  out_ref):
  mean = (parts_ref[0] + parts_ref[1]) * dinv_ref[...]
  o = (jnp.dot(mean, wlt_ref[...], preferred_element_type=jnp.float32)
       + b_ref[0:1, :]
       + jnp.dot(h_ref[...], wrt_ref[...], preferred_element_type=jnp.float32))
  out_ref[...] = jnp.maximum(o, 0.0)


_parts_spec = pl.BlockSpec((2, _R, _D), lambda i: (0, i, 0))
_rows_spec = pl.BlockSpec((_R, _D), lambda i: (i, 0))
_w_spec = pl.BlockSpec((_D, _D), lambda i: (0, 0))
_b_spec = pl.BlockSpec((8, _D), lambda i: (0, 0))

_dense1 = pl.pallas_call(
    _dense1_body,
    grid=(_GRID,),
    in_specs=[_parts_spec, _parts_spec,
              _rows_spec, _w_spec, _b_spec, _w_spec],
    out_specs=[_rows_spec, _rows_spec],
    out_shape=[jax.ShapeDtypeStruct((_N, _D), jnp.float32),
               jax.ShapeDtypeStruct((_N, _D), jnp.float32)],
)

_dense23 = pl.pallas_call(
    _dense23_body,
    grid=(_GRID,),
    in_specs=[_parts_spec, _rows_spec, _rows_spec, _w_spec, _b_spec, _w_spec],
    out_specs=_rows_spec,
    out_shape=jax.ShapeDtypeStruct((_N, _D), jnp.float32),
)


def _topk_body(h_ref, batch_ref, sel_ref, d_ref):
  xl = h_ref[:, _D - 1:_D]
  g_iota = lax.broadcasted_iota(jnp.int32, (_MP, _B), 1)
  member = batch_ref[...] == g_iota
  counts = jnp.sum(member.astype(jnp.int32), axis=0, keepdims=True)
  d_ref[...] = jnp.where(member, jnp.broadcast_to(xl, (_MP, _B)), _NEG)
  row_iota = lax.broadcasted_iota(jnp.int32, (_MP, _B), 0)

  def step(k, carry):
    dm = d_ref[...]
    m = jnp.max(dm, axis=0, keepdims=True)
    cand = jnp.where(dm == m, row_iota, _BIGI)
    idx = jnp.min(cand, axis=0, keepdims=True)
    valid = k < counts
    sel_ref[pl.ds(k, 1), :] = jnp.where(valid, idx, _ZROW)
    d_ref[...] = jnp.where(row_iota == idx, _NEG, dm)
    return carry

  lax.fori_loop(0, _K, step, 0)
  sel_ref[_K:_K + 2, :] = jnp.full((2, _B), _ZROW, jnp.int32)


_topk = pl.pallas_call(
    _topk_body,
    in_specs=[pl.BlockSpec((_MP, _D), lambda: (0, 0)),
              pl.BlockSpec((_MP, 1), lambda: (0, 0))],
    out_specs=pl.BlockSpec((_K + 2, _B), lambda: (0, 0)),
    out_shape=jax.ShapeDtypeStruct((_K + 2, _B), jnp.int32),
    scratch_shapes=[pltpu.VMEM((_MP, _B), jnp.float32)],
)


_CT = _K - 5 + 1


def _head_body(s_ref, wc_ref, bc_ref, w1_ref, b1_ref, w2_ref, b2_ref, out_ref):
  acc1 = jnp.zeros((_B, _D), jnp.float32)
  for t in range(_CT):
    ct = jnp.zeros((_B, 32), jnp.float32)
    for tau in range(5):
      sk = s_ref[(t + tau) * _B:(t + tau + 1) * _B, :]
      ct = ct + jnp.dot(sk, wc_ref[tau], preferred_element_type=jnp.float32)
    ct = jnp.maximum(ct + bc_ref[0:1, :], 0.0)
    acc1 = acc1 + jnp.dot(ct, w1_ref[t], preferred_element_type=jnp.float32)
  l1 = jnp.maximum(acc1 + b1_ref[0:1, :], 0.0)
  logits = jnp.dot(l1, w2_ref[...], preferred_element_type=jnp.float32) \
      + b2_ref[0:1, :]
  m = jnp.max(logits, axis=-1, keepdims=True)
  lse = m + jnp.log(jnp.sum(jnp.exp(logits - m), axis=-1, keepdims=True))
  out_ref[...] = logits - lse


_head = pl.pallas_call(
    _head_body,
    in_specs=[pl.BlockSpec((_NW * _CH, _D), lambda: (0, 0)),
              pl.BlockSpec((5, _D, 32), lambda: (0, 0, 0)),
              pl.BlockSpec((8, 32), lambda: (0, 0)),
              pl.BlockSpec((_CT, 32, _D), lambda: (0, 0, 0)),
              pl.BlockSpec((8, _D), lambda: (0, 0)),
              pl.BlockSpec((_D, _D), lambda: (0, 0)),
              pl.BlockSpec((8, _D), lambda: (0, 0))],
    out_specs=pl.BlockSpec((_B, _D), lambda: (0, 0)),
    out_shape=jax.ShapeDtypeStruct((_B, _D), jnp.float32),
)


def kernel(x, edge_index, batch, W_l1, b_l1, W_r1, W_l2, b_l2, W_r2,
           W_l3, b_l3, W_r3, Wc, bc, W1, b1, W2, b2):
  src = edge_index[0].astype(jnp.int32)
  dst = edge_index[1].astype(jnp.int32)
  e = src.shape[0]
  nbt = 2 * (-(-e // (_NW * _EIDB * _ECH)))
  e_pad = _NS * nbt * _EIDB * _ECH
  nb0 = nbt * _AGG_FRAC0 // 100
  nb1 = nbt - nb0
  src_flat = jnp.concatenate([src, jnp.zeros((e_pad - e,), jnp.int32)])
  dst_flat = jnp.concatenate(
      [dst, jnp.full((e_pad - e,), _NP - 1, jnp.int32)])
  zeros_acc = jnp.zeros((_NP, _D), jnp.float32)
  ones_rows = jnp.ones((_CH, _D), jnp.float32)

  agg = _make_sc_agg(nb0, nb1)
  degk = _make_sc_deg(e_pad // (_NW * _CH))
  src2d = src_flat.reshape(-1, _ECH)
  dst2d = dst_flat.reshape(-1, _ECH)
  dst2d_deg = dst_flat.reshape(-1, _CH)

  bl1 = jnp.broadcast_to(b_l1, (8, _D))
  bl2 = jnp.broadcast_to(b_l2, (8, _D))
  bl3 = jnp.broadcast_to(b_l3, (8, _D))

  (degp,) = degk(dst2d_deg, zeros_acc, ones_rows)
  (parts1,) = agg(x, src2d, dst2d, zeros_acc)
  h1, dinv = _dense1(parts1.reshape(2, _NP, _D), degp.reshape(2, _NP, _D),
                     x, W_l1.T, bl1, W_r1.T)
  (parts2,) = agg(h1, src2d, dst2d, zeros_acc)
  h2 = _dense23(parts2.reshape(2, _NP, _D), dinv, h1, W_l2.T, bl2, W_r2.T)
  (parts3,) = agg(h2, src2d, dst2d, zeros_acc)
  h3 = _dense23(parts3.reshape(2, _NP, _D), dinv, h2, W_l3.T, bl3, W_r3.T)

  h3p = jnp.concatenate([h3, jnp.zeros((_MP - _N, _D), jnp.float32)])
  batchcol = jnp.concatenate(
      [batch.astype(jnp.int32),
       jnp.full((_MP - _N,), _PADG, jnp.int32)]).reshape(_MP, 1)
  sel = _topk(h3p, batchcol)
  (s_rows,) = _make_sc_gather()(h3p, sel.reshape(-1))

  wcr = jnp.transpose(Wc, (2, 1, 0))
  bc8 = jnp.broadcast_to(bc, (8, 32))
  w1r = jnp.transpose(W1.reshape(_D, 32, _CT), (2, 1, 0))
  b18 = jnp.broadcast_to(b1, (8, _D))
  w2t = jnp.zeros((_D, _D), jnp.float32).at[:, :10].set(W2.T)
  b2r = jnp.full((8, _D), -1.0e30, jnp.float32).at[:, :10].set(
      jnp.broadcast_to(b2, (8, 10)))
  out = _head(s_rows, wcr, bc8, w1r, b18, w2t, b2r)
  return out[:, :10]


# speedup vs baseline: 1.3450x; 1.3450x over previous
import functools

import jax
import jax.numpy as jnp
from jax import lax
from jax.experimental import pallas as pl
from jax.experimental.pallas import tpu as pltpu
from jax.experimental.pallas import tpu_sc as plsc

_N = 10000
_D = 128
_B = 128
_K = 30
_NC = 2
_NS = 16
_NW = _NC * _NS
_CH = 128
_NP = 10240
_RPT = _NP // _NS
_MP = 10112
_ZROW = _MP - 1
_NEG = -3.0e38
_BIGI = 1 << 29
_PADG = 1 << 20
_IDB = 16
_ECH = 64
_EIDB = 16
_NBUF = 4
_AGG_FRAC0 = 95


def _mesh():
  return plsc.VectorSubcoreMesh(core_axis_name="c", subcore_axis_name="s",
                                num_cores=_NC, num_subcores=_NS)


@functools.cache
def _make_sc_agg(nb0: int, nb1: int):
  cpt0 = nb0 * _EIDB
  cpt1 = nb1 * _EIDB
  scratch = [
      pltpu.VMEM((_EIDB, _ECH), jnp.int32),
      pltpu.VMEM((_EIDB, _ECH), jnp.int32),
      pltpu.VMEM_SHARED((_NP, _D), jnp.float32),
  ] + [pltpu.VMEM((_ECH, _D), jnp.float32)] * _NBUF \
    + [pltpu.SemaphoreType.DMA] * (2 * _NBUF)

  def body(h_hbm, src_hbm, dst_hbm, zeros_hbm, out_acc,
           src_v, dst_v, acc_sh, *bufs):
    rows = bufs[:_NBUF]
    gsem = bufs[_NBUF:2 * _NBUF]
    ssem = bufs[2 * _NBUF:]
    c = lax.axis_index("c")
    s = lax.axis_index("s")
    pltpu.sync_copy(zeros_hbm.at[pl.ds(s * _RPT, _RPT)],
                    acc_sh.at[pl.ds(s * _RPT, _RPT)])
    nblk = jnp.where(c == 0, nb0, nb1)
    t0 = jnp.where(c == 0, s * cpt0, _NS * cpt0 + s * cpt1)
    plsc.subcore_barrier()

    def block(bi, carry):
      pltpu.sync_copy(src_hbm.at[pl.ds(t0 + bi * _EIDB, _EIDB)], src_v)
      pltpu.sync_copy(dst_hbm.at[pl.ds(t0 + bi * _EIDB, _EIDB)], dst_v)
      gd = [None] * _NBUF
      sd = [None] * _NBUF
      for p in range(_NBUF - 1):
        gd[p] = pltpu.async_copy(h_hbm.at[src_v.at[p]], rows[p], gsem[p])
      for j in range(_EIDB):
        b = j % _NBUF
        nxt = j + _NBUF - 1
        if nxt < _EIDB:
          nb = nxt % _NBUF
          if sd[nb] is not None:
            sd[nb].wait()
          gd[nb] = pltpu.async_copy(h_hbm.at[src_v.at[nxt]],
                                    rows[nb], gsem[nb])
        gd[b].wait()
        sd[b] = pltpu.async_copy(rows[b], acc_sh.at[dst_v.at[j]],
                                 ssem[b], add=True)
      for p in range(_NBUF):
        sd[(_EIDB - _NBUF + p) % _NBUF].wait()
      return carry

    lax.fori_loop(0, nblk, block, 0)
    plsc.subcore_barrier()
    off = c * _NP + s * _RPT
    pltpu.sync_copy(acc_sh.at[pl.ds(s * _RPT, _RPT)],
                    out_acc.at[pl.ds(off, _RPT)])

  return pl.kernel(
      body,
      out_type=[jax.ShapeDtypeStruct((_NC * _NP, _D), jnp.float32)],
      mesh=_mesh(), scratch_types=scratch)


@functools.cache
def _make_sc_deg(chunks_per_tile: int):
  scratch = [
      pltpu.VMEM((_IDB, _CH), jnp.int32),
      pltpu.VMEM((_CH, _D), jnp.float32),
      pltpu.VMEM_SHARED((_NP, _D), jnp.float32),
      pltpu.SemaphoreType.DMA,
  ]

  def body(dst_hbm, zeros_hbm, ones_hbm, out_acc, dst_v, ones_v, acc_sh, sem):
    c = lax.axis_index("c")
    s = lax.axis_index("s")
    pltpu.sync_copy(zeros_hbm.at[pl.ds(s * _RPT, _RPT)],
                    acc_sh.at[pl.ds(s * _RPT, _RPT)])
    pltpu.sync_copy(ones_hbm, ones_v)
    t0 = (c * _NS + s) * chunks_per_tile
    plsc.subcore_barrier()

    def block(bi, carry):
      pltpu.sync_copy(dst_hbm.at[pl.ds(t0 + bi * _IDB, _IDB)], dst_v)
      descs = [pltpu.async_copy(ones_v, acc_sh.at[dst_v.at[j]], sem, add=True)
               for j in range(_IDB)]
      for d in descs:
        d.wait()
      return carry

    lax.fori_loop(0, chunks_per_tile // _IDB, block, 0)
    plsc.subcore_barrier()
    off = c * _NP + s * _RPT
    pltpu.sync_copy(acc_sh.at[pl.ds(s * _RPT, _RPT)],
                    out_acc.at[pl.ds(off, _RPT)])

  return pl.kernel(
      body,
      out_type=[jax.ShapeDtypeStruct((_NC * _NP, _D), jnp.float32)],
      mesh=_mesh(), scratch_types=scratch)


def _sc_gather_body(h_hbm, sel_hbm, out_hbm, idx_v, rows_v, sem):
  w = lax.axis_index("c") * _NS + lax.axis_index("s")
  pltpu.sync_copy(sel_hbm.at[pl.ds(w * _CH, _CH)], idx_v)
  pltpu.async_copy(h_hbm.at[idx_v], rows_v, sem).wait()
  pltpu.sync_copy(rows_v, out_hbm.at[pl.ds(w * _CH, _CH)])


@functools.cache
def _make_sc_gather():
  return pl.kernel(
      _sc_gather_body,
      out_type=[jax.ShapeDtypeStruct((_NW * _CH, _D), jnp.float32)],
      mesh=_mesh(),
      scratch_types=[
          pltpu.VMEM((_CH,), jnp.int32),
          pltpu.VMEM((_CH, _D), jnp.float32),
          pltpu.SemaphoreType.DMA,
      ])


_R = 400
_GRID = _N // _R


def _dense1_body(parts_ref, degp_ref, h_ref, wlt_ref, b_ref, wrt_ref,
                 out_ref, dinv_ref):
  agg = parts_ref[0] + parts_ref[1]
  deg = degp_ref[0, :, 0:1] + degp_ref[1, :, 0:1]
  dinv = 1.0 / jnp.maximum(deg, 1.0)
  mean = agg * dinv
  o = (jnp.dot(mean, wlt_ref[...], preferred_element_type=jnp.float32)
       + b_ref[0:1, :]
       + jnp.dot(h_ref[...], wrt_ref[...], preferred_element_type=jnp.float32))
  out_ref[...] = jnp.maximum(o, 0.0)
  dinv_ref[...] = jnp.broadcast_to(dinv, (_R, _D))


def _dense23_body(parts_ref, dinv_ref, h_ref, wlt_ref, b_ref, wrt_ref,
                  out_ref):
  mean = (parts_ref[0] + parts_ref[1]) * dinv_ref[...]
  o = (jnp.dot(mean, wlt_ref[...], preferred_element_type=jnp.float32)
       + b_ref[0:1, :]
       + jnp.dot(h_ref[...], wrt_ref[...], preferred_element_type=jnp.float32))
  out_ref[...] = jnp.maximum(o, 0.0)


_parts_spec = pl.BlockSpec((2, _R, _D), lambda i: (0, i, 0))
_rows_spec = pl.BlockSpec((_R, _D), lambda i: (i, 0))
_w_spec = pl.BlockSpec((_D, _D), lambda i: (0, 0))
_b_spec = pl.BlockSpec((8, _D), lambda i: (0, 0))

_dense1 = pl.pallas_call(
    _dense1_body,
    grid=(_GRID,),
    in_specs=[_parts_spec, _parts_spec,
              _rows_spec, _w_spec, _b_spec, _w_spec],
    out_specs=[_rows_spec, _rows_spec],
    out_shape=[jax.ShapeDtypeStruct((_N, _D), jnp.float32),
               jax.ShapeDtypeStruct((_N, _D), jnp.float32)],
)

_dense23 = pl.pallas_call(
    _dense23_body,
    grid=(_GRID,),
    in_specs=[_parts_spec, _rows_spec, _rows_spec, _w_spec, _b_spec, _w_spec],
    out_specs=_rows_spec,
    out_shape=jax.ShapeDtypeStruct((_N, _D), jnp.float32),
)


def _topk_body(h_ref, batch_ref, sel_ref, d_ref):
  xl = h_ref[:, _D - 1:_D]
  g_iota = lax.broadcasted_iota(jnp.int32, (_MP, _B), 1)
  member = batch_ref[...] == g_iota
  counts = jnp.sum(member.astype(jnp.int32), axis=0, keepdims=True)
  d_ref[...] = jnp.where(member, jnp.broadcast_to(xl, (_MP, _B)), _NEG)
  row_iota = lax.broadcasted_iota(jnp.int32, (_MP, _B), 0)

  def step(k, carry):
    dm = d_ref[...]
    m = jnp.max(dm, axis=0, keepdims=True)
    cand = jnp.where(dm == m, row_iota, _BIGI)
    idx = jnp.min(cand, axis=0, keepdims=True)
    valid = k < counts
    sel_ref[pl.ds(k, 1), :] = jnp.where(valid, idx, _ZROW)
    d_ref[...] = jnp.where(row_iota == idx, _NEG, dm)
    return carry

  lax.fori_loop(0, _K, step, 0)
  sel_ref[_K:_K + 2, :] = jnp.full((2, _B), _ZROW, jnp.int32)


_topk = pl.pallas_call(
    _topk_body,
    in_specs=[pl.BlockSpec((_MP, _D), lambda: (0, 0)),
              pl.BlockSpec((_MP, 1), lambda: (0, 0))],
    out_specs=pl.BlockSpec((_K + 2, _B), lambda: (0, 0)),
    out_shape=jax.ShapeDtypeStruct((_K + 2, _B), jnp.int32),
    scratch_shapes=[pltpu.VMEM((_MP, _B), jnp.float32)],
)


_CT = _K - 5 + 1


def _head_body(s_ref, wc_ref, bc_ref, w1_ref, b1_ref, w2_ref, b2_ref, out_ref):
  acc1 = jnp.zeros((_B, _D), jnp.float32)
  for t in range(_CT):
    ct = jnp.zeros((_B, 32), jnp.float32)
    for tau in range(5):
      sk = s_ref[(t + tau) * _B:(t + tau + 1) * _B, :]
      ct = ct + jnp.dot(sk, wc_ref[tau], preferred_element_type=jnp.float32)
    ct = jnp.maximum(ct + bc_ref[0:1, :], 0.0)
    acc1 = acc1 + jnp.dot(ct, w1_ref[t], preferred_element_type=jnp.float32)
  l1 = jnp.maximum(acc1 + b1_ref[0:1, :], 0.0)
  logits = jnp.dot(l1, w2_ref[...], preferred_element_type=jnp.float32) \
      + b2_ref[0:1, :]
  m = jnp.max(logits, axis=-1, keepdims=True)
  lse = m + jnp.log(jnp.sum(jnp.exp(logits - m), axis=-1, keepdims=True))
  out_ref[...] = logits - lse


_head = pl.pallas_call(
    _head_body,
    in_specs=[pl.BlockSpec((_NW * _CH, _D), lambda: (0, 0)),
              pl.BlockSpec((5, _D, 32), lambda: (0, 0, 0)),
              pl.BlockSpec((8, 32), lambda: (0, 0)),
              pl.BlockSpec((_CT, 32, _D), lambda: (0, 0, 0)),
              pl.BlockSpec((8, _D), lambda: (0, 0)),
              pl.BlockSpec((_D, _D), lambda: (0, 0)),
              pl.BlockSpec((8, _D), lambda: (0, 0))],
    out_specs=pl.BlockSpec((_B, _D), lambda: (0, 0)),
    out_shape=jax.ShapeDtypeStruct((_B, _D), jnp.float32),
)


def kernel(x, edge_index, batch, W_l1, b_l1, W_r1, W_l2, b_l2, W_r2,
           W_l3, b_l3, W_r3, Wc, bc, W1, b1, W2, b2):
  src = edge_index[0].astype(jnp.int32)
  dst = edge_index[1].astype(jnp.int32)
  e = src.shape[0]
  nbt = 2 * (-(-e // (_NW * _EIDB * _ECH)))
  e_pad = _NS * nbt * _EIDB * _ECH
  nb0 = nbt * _AGG_FRAC0 // 100
  nb1 = nbt - nb0
  src_flat = jnp.concatenate([src, jnp.zeros((e_pad - e,), jnp.int32)])
  dst_flat = jnp.concatenate(
      [dst, jnp.full((e_pad - e,), _NP - 1, jnp.int32)])
  zeros_acc = jnp.zeros((_NP, _D), jnp.float32)
  ones_rows = jnp.ones((_CH, _D), jnp.float32)

  agg = _make_sc_agg(nb0, nb1)
  degk = _make_sc_deg(e_pad // (_NW * _CH))
  src2d = src_flat.reshape(-1, _ECH)
  dst2d = dst_flat.reshape(-1, _ECH)
  dst2d_deg = dst_flat.reshape(-1, _CH)

  bl1 = jnp.broadcast_to(b_l1, (8, _D))
  bl2 = jnp.broadcast_to(b_l2, (8, _D))
  bl3 = jnp.broadcast_to(b_l3, (8, _D))

  (degp,) = degk(dst2d_deg, zeros_acc, ones_rows)
  (parts1,) = agg(x, src2d, dst2d, zeros_acc)
  h1, dinv = _dense1(parts1.reshape(2, _NP, _D), degp.reshape(2, _NP, _D),
                     x, W_l1.T, bl1, W_r1.T)
  (parts2,) = agg(h1, src2d, dst2d, zeros_acc)
  h2 = _dense23(parts2.reshape(2, _NP, _D), dinv, h1, W_l2.T, bl2, W_r2.T)
  (parts3,) = agg(h2, src2d, dst2d, zeros_acc)
  h3 = _dense23(parts3.reshape(2, _NP, _D), dinv, h2, W_l3.T, bl3, W_r3.T)

  h3p = jnp.concatenate([h3, jnp.zeros((_MP - _N, _D), jnp.float32)])
  batchcol = jnp.concatenate(
      [batch.astype(jnp.int32),
       jnp.full((_MP - _N,), _PADG, jnp.int32)]).reshape(_MP, 1)
  sel = _topk(h3p, batchcol)
  (s_rows,) = _make_sc_gather()(h3p, sel.reshape(-1))

  wcr = jnp.transpose(Wc, (2, 1, 0))
  bc8 = jnp.broadcast_to(bc, (8, 32))
  w1r = jnp.transpose(W1.reshape(_D, 32, _CT), (2, 1, 0))
  b18 = jnp.broadcast_to(b1, (8, _D))
  w2t = jnp.zeros((_D, _D), jnp.float32).at[:, :10].set(W2.T)
  b2r = jnp.full((8, _D), -1.0e30, jnp.float32).at[:, :10].set(
      jnp.broadcast_to(b2, (8, 10)))
  out = _head(s_rows, wcr, bc8, w1r, b18, w2t, b2r)
  return out[:, :10]
